# Initial kernel scaffold; baseline (speedup 1.0000x reference)
import jax
import jax.numpy as jnp
from jax.experimental import pallas as pl


def kernel(x_s, edge_index_s, x_s_batch, x_t, edge_index_t, x_t_batch, y, W1, b1, W2, b2):
    raise NotImplementedError("write your pallas kernel here")


# speedup vs baseline: 41.3288x; 41.3288x over previous
import functools

import jax
import jax.numpy as jnp
from jax import lax
from jax.experimental import pallas as pl
from jax.experimental.pallas import tpu as pltpu
from jax.experimental.pallas import tpu_sc as plsc

N = 10000
E = 320000
D = 128
H = 32
G = 64
C = 96

NPAD = 10240
ROWS_W = NPAD // 16
CH = 128
NCHUNK = 157
EPS = NCHUNK * CH
EPAD = EPS * 16

_mesh = plsc.VectorSubcoreMesh(core_axis_name="c", subcore_axis_name="s")
_sc_params = pltpu.CompilerParams(use_tc_tiling_on_sc=False)


@functools.partial(
    pl.kernel,
    out_type=jax.ShapeDtypeStruct((2, NPAD), jnp.float32),
    mesh=_mesh,
    scratch_types=[
        pltpu.VMEM((NCHUNK, CH), jnp.int32),
        pltpu.VMEM((CH,), jnp.float32),
        pltpu.VMEM((ROWS_W,), jnp.float32),
        pltpu.VMEM_SHARED((NPAD,), jnp.float32),
    ],
    compiler_params=_sc_params,
)
def _deg_kernel(dst_hbm, deg_hbm, dst_v, ones_v, zeros_v, deg_sp):
    cid = lax.axis_index("c")
    sid = lax.axis_index("s")
    for i in range(CH // 16):
        ones_v[pl.ds(i * 16, 16)] = jnp.full((16,), 1.0, jnp.float32)
    for i in range(ROWS_W // 16):
        zeros_v[pl.ds(i * 16, 16)] = jnp.zeros((16,), jnp.float32)
    pltpu.sync_copy(zeros_v, deg_sp.at[pl.ds(sid * ROWS_W, ROWS_W)])
    pltpu.sync_copy(dst_hbm.at[cid, sid], dst_v)
    plsc.subcore_barrier()

    def body(i, carry):
        pltpu.sync_copy(ones_v, deg_sp.at[dst_v.at[i]], add=True)
        return carry

    lax.fori_loop(0, NCHUNK, body, 0)
    plsc.subcore_barrier()
    pltpu.sync_copy(deg_sp.at[pl.ds(sid * ROWS_W, ROWS_W)],
                    deg_hbm.at[cid, pl.ds(sid * ROWS_W, ROWS_W)])


@functools.partial(
    pl.kernel,
    out_type=jax.ShapeDtypeStruct((2, NPAD, H), jnp.float32),
    mesh=_mesh,
    scratch_types=[
        pltpu.VMEM((NCHUNK, CH), jnp.int32),
        pltpu.VMEM((NCHUNK, CH), jnp.int32),
        pltpu.VMEM((CH, H), jnp.float32),
        pltpu.VMEM_SHARED((NPAD, H), jnp.float32),
        pltpu.SemaphoreType.DMA,
    ],
    compiler_params=_sc_params,
)
def _agg_kernel(u_hbm, src_hbm, dst_hbm, acc_hbm, src_v, dst_v, rows_v,
                acc_sp, sem):
    cid = lax.axis_index("c")
    sid = lax.axis_index("s")
    for r in range(CH):
        for j in range(H // 16):
            rows_v[r, pl.ds(j * 16, 16)] = jnp.zeros((16,), jnp.float32)
    for k in range(ROWS_W // CH):
        pltpu.sync_copy(rows_v, acc_sp.at[pl.ds(sid * ROWS_W + k * CH, CH)])
    pltpu.sync_copy(src_hbm.at[cid, sid], src_v)
    pltpu.sync_copy(dst_hbm.at[cid, sid], dst_v)
    plsc.subcore_barrier()

    def body(i, carry):
        pltpu.async_copy(u_hbm.at[src_v.at[i]], rows_v, sem).wait()
        pltpu.sync_copy(rows_v, acc_sp.at[dst_v.at[i]], add=True)
        return carry

    lax.fori_loop(0, NCHUNK, body, 0)
    plsc.subcore_barrier()
    pltpu.sync_copy(acc_sp.at[pl.ds(sid * ROWS_W, ROWS_W)],
                    acc_hbm.at[cid, pl.ds(sid * ROWS_W, ROWS_W)])


def _mid_body(x_ref, w1_ref, deg_ref, u_ref, dinv_ref):
    d = deg_ref[...] + 1.0
    dv = lax.rsqrt(d)
    xw = jnp.dot(x_ref[...], w1_ref[...], preferred_element_type=jnp.float32)
    u_ref[...] = xw * dv
    dinv_ref[...] = dv


def _tc_mid(X, W1, deg_flat):
    blk = 2048
    grid = (2 * NPAD) // blk
    return pl.pallas_call(
        _mid_body,
        grid=(grid,),
        in_specs=[
            pl.BlockSpec((blk, D), lambda i: (i, 0)),
            pl.BlockSpec((D, H), lambda i: (0, 0)),
            pl.BlockSpec((blk, 1), lambda i: (i, 0)),
        ],
        out_specs=[
            pl.BlockSpec((blk, H), lambda i: (i, 0)),
            pl.BlockSpec((blk, 1), lambda i: (i, 0)),
        ],
        out_shape=[
            jax.ShapeDtypeStruct((2 * NPAD, H), jnp.float32),
            jax.ShapeDtypeStruct((2 * NPAD, 1), jnp.float32),
        ],
    )(X, W1, deg_flat)


def _final_body(acc_ref, u_ref, dinv_ref, b1_ref, batch_ref, w2_ref, b2_ref,
                out_ref):
    embs = []
    u_all = u_ref[...]
    dv_all = dinv_ref[...]
    for c in range(2):
        a_c = acc_ref[c]
        u_c = u_all[c * NPAD:(c + 1) * NPAD]
        dv_c = dv_all[c * NPAD:(c + 1) * NPAD]
        h = jnp.tanh(dv_c * (a_c + u_c) + b1_ref[...])
        b_c = batch_ref[c]
        iota = lax.broadcasted_iota(jnp.int32, (NPAD, G), 1)
        M = (b_c == iota).astype(jnp.float32)
        sums = lax.dot_general(M, h, (((0,), (0,)), ((), ())),
                               preferred_element_type=jnp.float32)
        ones = jnp.ones((NPAD, 1), jnp.float32)
        cnt = lax.dot_general(M, ones, (((0,), (0,)), ((), ())),
                              preferred_element_type=jnp.float32)
        pooled = sums / jnp.maximum(cnt, 1.0)
        embs.append(jnp.tanh(pooled))
    embedding = jnp.concatenate(embs, axis=1)
    out_ref[...] = (jnp.dot(embedding, w2_ref[...],
                            preferred_element_type=jnp.float32) + b2_ref[...])


def _tc_final(acc, U, dinv, b1, batch2, W2, b2):
    return pl.pallas_call(
        _final_body,
        out_shape=jax.ShapeDtypeStruct((G, C), jnp.float32),
    )(acc, U, dinv, b1.reshape(1, H), batch2, W2, b2.reshape(1, C))


def _pad_edges(ei, offset):
    src = jnp.concatenate([ei[0], jnp.full((EPAD - E,), N, jnp.int32)])
    dst = jnp.concatenate([ei[1], jnp.full((EPAD - E,), N, jnp.int32)])
    return (src + offset).reshape(16, NCHUNK, CH), dst.reshape(16, NCHUNK, CH)


def kernel(x_s, edge_index_s, x_s_batch, x_t, edge_index_t, x_t_batch, y, W1,
           b1, W2, b2):
    zrows = jnp.zeros((NPAD - N, D), jnp.float32)
    X = jnp.concatenate([x_s, zrows, x_t, zrows])

    src_s, dst_s = _pad_edges(edge_index_s, 0)
    src_t, dst_t = _pad_edges(edge_index_t, NPAD)
    src_comb = jnp.stack([src_s, src_t])
    dst_comb = jnp.stack([dst_s, dst_t])

    bpad = jnp.full((NPAD - N,), G, jnp.int32)
    batch2 = jnp.stack([jnp.concatenate([x_s_batch, bpad]),
                        jnp.concatenate([x_t_batch, bpad])])[..., None]

    deg = _deg_kernel(dst_comb)
    U, dinv = _tc_mid(X, W1, deg.reshape(2 * NPAD, 1))
    acc = _agg_kernel(U, src_comb, dst_comb)
    return _tc_final(acc, U, dinv, b1, batch2, W2, b2)
